# Initial kernel scaffold; baseline (speedup 1.0000x reference)
import jax
import jax.numpy as jnp
from jax.experimental import pallas as pl


def kernel(x):
    raise NotImplementedError("write your pallas kernel here")


# speedup vs baseline: 1.5715x; 1.5715x over previous
import functools

import jax
import jax.numpy as jnp
from jax import lax
from jax.experimental import pallas as pl

_B, _H, _L = 16, 512, 64
_HB = 128
_NEG = float("-inf")


def _pattern(dtype):
    r = lax.broadcasted_iota(jnp.int32, (_L, _L), 0)
    c = lax.broadcasted_iota(jnp.int32, (_L, _L), 1)
    d = c - r
    pat = (d >= 0) & (d <= 15)
    pat |= (r % 2 == 0) & (d % 2 == 1) & (d >= 17) & (d <= 31)
    pat |= (r % 4 == 0) & (d % 4 == 3) & (d >= 35)
    return pat, pat.astype(dtype)


def _map_kernel(x_ref, out_ref):
    xb = x_ref[...]
    pat, _ = _pattern(xb.dtype)
    r = lax.broadcasted_iota(jnp.int32, (_L, _L), 0)
    c = lax.broadcasted_iota(jnp.int32, (_L, _L), 1)
    a = jnp.where((c >= r)[None], xb[:, None, :], _NEG)
    s = 1
    while s < _L:
        shifted = jnp.concatenate(
            [jnp.full((_HB, _L, s), _NEG, xb.dtype), a[..., :-s]], axis=-1)
        a = jnp.maximum(a, shifted)
        s *= 2
    out_ref[...] = jnp.where(pat[None], a, 0.0)


def _mask_kernel(out_ref):
    _, patf = _pattern(out_ref.dtype)
    out_ref[...] = jnp.broadcast_to(patf[None, None], out_ref.shape)


@jax.jit
def kernel(x):
    rows = _B * _H
    x2 = x.reshape(rows, _L)
    grid = (rows // _HB,)
    ori_h = pl.pallas_call(
        _map_kernel,
        grid=grid,
        in_specs=[pl.BlockSpec((_HB, _L), lambda j: (j, 0))],
        out_specs=pl.BlockSpec((_HB, _L, _L), lambda j: (j, 0, 0)),
        out_shape=jax.ShapeDtypeStruct((rows, _L, _L), x.dtype),
    )(x2)
    ori_mask = pl.pallas_call(
        _mask_kernel,
        out_shape=jax.ShapeDtypeStruct((_B, 1, _L, _L), x.dtype),
    )()
    return ori_h.reshape(_B, _H, _L, _L), ori_mask


# speedup vs baseline: 1.5805x; 1.0058x over previous
import functools

import jax
import jax.numpy as jnp
from jax import lax
from jax.experimental import pallas as pl

_B, _H, _L = 16, 512, 64
_HB = 128
_NEG = float("-inf")


def _rc_packed():
    q = lax.broadcasted_iota(jnp.int32, (_L // 2, 2 * _L), 0)
    p = lax.broadcasted_iota(jnp.int32, (_L // 2, 2 * _L), 1)
    r = 2 * q + (p // _L)
    c = p % _L
    return r, c, p % _L


def _pattern(r, c, dtype):
    d = c - r
    pat = (d >= 0) & (d <= 15)
    pat |= (r % 2 == 0) & (d % 2 == 1) & (d >= 17) & (d <= 31)
    pat |= (r % 4 == 0) & (d % 4 == 3) & (d >= 35)
    return pat, pat.astype(dtype)


def _map_kernel(x_ref, out_ref):
    xb = x_ref[...]
    r, c, cm = _rc_packed()
    pat, _ = _pattern(r, c, xb.dtype)
    xx = jnp.concatenate([xb, xb], axis=-1)
    a = jnp.where((c >= r)[None], xx[:, None, :], _NEG)
    s = 1
    while s < _L:
        shifted = jnp.where((cm < s)[None], _NEG, jnp.roll(a, s, axis=-1))
        a = jnp.maximum(a, shifted)
        s *= 2
    out_ref[...] = jnp.where(pat[None], a, 0.0)


def _mask_kernel(out_ref):
    r = lax.broadcasted_iota(jnp.int32, (_L, _L), 0)
    c = lax.broadcasted_iota(jnp.int32, (_L, _L), 1)
    _, patf = _pattern(r, c, out_ref.dtype)
    out_ref[...] = jnp.broadcast_to(patf[None, None], out_ref.shape)


@jax.jit
def kernel(x):
    rows = _B * _H
    x2 = x.reshape(rows, _L)
    grid = (rows // _HB,)
    ori_h = pl.pallas_call(
        _map_kernel,
        grid=grid,
        in_specs=[pl.BlockSpec((_HB, _L), lambda j: (j, 0))],
        out_specs=pl.BlockSpec((_HB, _L // 2, 2 * _L), lambda j: (j, 0, 0)),
        out_shape=jax.ShapeDtypeStruct((rows, _L // 2, 2 * _L), x.dtype),
    )(x2)
    ori_mask = pl.pallas_call(
        _mask_kernel,
        out_shape=jax.ShapeDtypeStruct((_B, 1, _L, _L), x.dtype),
    )()
    return ori_h.reshape(_B, _H, _L, _L), ori_mask
